# baseline (speedup 1.0000x reference)
import functools

import jax
import jax.numpy as jnp
from jax import lax
from jax.experimental import pallas as pl
from jax.experimental.pallas import tpu as pltpu
from jax.experimental.pallas import tpu_sc as plsc

BATCH = 4096
SEQ = 200
EMBED_DIM = 64

_INFO = plsc.get_sparse_core_info()
NC = _INFO.num_cores
NS = _INFO.num_subcores
NW = NC * NS
ROWS_PER_W = BATCH // NW
CHUNK = 100
CHUNKS_PER_ROW = SEQ // CHUNK
CHUNKS_PER_W = ROWS_PER_W * CHUNKS_PER_ROW
LANES = 16
VECS = EMBED_DIM // LANES


def _encoder_body(ids_hbm, table_hbm, out_hbm, idx_v, rows_v, out_v, sem):
    wid = lax.axis_index("s") * NC + lax.axis_index("c")

    pltpu.sync_copy(ids_hbm.at[wid], idx_v)

    inv_seq = jnp.float32(1.0 / SEQ)

    def row_body(b, _):
        c0 = b * CHUNKS_PER_ROW
        cp0 = pltpu.async_copy(
            table_hbm.at[idx_v.at[c0]], rows_v.at[pl.ds(0, CHUNK)], sem)
        cp1 = pltpu.async_copy(
            table_hbm.at[idx_v.at[c0 + 1]], rows_v.at[pl.ds(CHUNK, CHUNK)], sem)
        cp0.wait()
        cp1.wait()

        def acc_body(r, carry):
            return tuple(
                carry[k] + rows_v[r, pl.ds(k * LANES, LANES)]
                for k in range(VECS)
            )

        zeros = tuple(jnp.zeros((LANES,), jnp.float32) for _ in range(VECS))
        acc = lax.fori_loop(0, SEQ, acc_body, zeros, unroll=2)
        for k in range(VECS):
            out_v[b, pl.ds(k * LANES, LANES)] = acc[k] * inv_seq
        return 0

    lax.fori_loop(0, ROWS_PER_W, row_body, 0)

    pltpu.sync_copy(out_v, out_hbm.at[pl.ds(wid * ROWS_PER_W, ROWS_PER_W)])


_encoder = pl.kernel(
    _encoder_body,
    out_type=jax.ShapeDtypeStruct((BATCH, EMBED_DIM), jnp.float32),
    mesh=plsc.VectorSubcoreMesh(core_axis_name="c", subcore_axis_name="s"),
    scratch_types=[
        pltpu.VMEM((CHUNKS_PER_W, CHUNK), jnp.int32),
        pltpu.VMEM((SEQ, EMBED_DIM), jnp.float32),
        pltpu.VMEM((ROWS_PER_W, EMBED_DIM), jnp.float32),
        pltpu.SemaphoreType.DMA,
    ],
    compiler_params=pltpu.CompilerParams(use_tc_tiling_on_sc=False),
)


@jax.jit
def kernel(text_ids, table):
    ids = text_ids.astype(jnp.int32).reshape(NW, CHUNKS_PER_W, CHUNK)
    return _encoder(ids, table)


# speedup vs baseline: 1.1988x; 1.1988x over previous
import jax
import jax.numpy as jnp
from jax import lax
from jax.experimental import pallas as pl
from jax.experimental.pallas import tpu as pltpu
from jax.experimental.pallas import tpu_sc as plsc

BATCH = 4096
SEQ = 200
EMBED_DIM = 64

_INFO = plsc.get_sparse_core_info()
NC = _INFO.num_cores
NS = _INFO.num_subcores
NW = NC * NS
ROWS_PER_W = BATCH // NW
CHUNK = 100
CHUNKS_PER_ROW = SEQ // CHUNK
CHUNKS_PER_W = ROWS_PER_W * CHUNKS_PER_ROW
LANES = 16
VECS = EMBED_DIM // LANES
RING_ROWS = 4
NBUF = RING_ROWS * CHUNKS_PER_ROW


def _encoder_body(ids_hbm, table_hbm, out_hbm, idx_v, rows_v, out_v, sems):
    wid = lax.axis_index("s") * NC + lax.axis_index("c")

    pltpu.sync_copy(ids_hbm.at[wid], idx_v)

    inv_seq = jnp.float32(1.0 / SEQ)

    def start_row(b, bufs):
        for h in range(CHUNKS_PER_ROW):
            pltpu.async_copy(
                table_hbm.at[idx_v.at[b * CHUNKS_PER_ROW + h]],
                rows_v.at[bufs[h]],
                sems.at[bufs[h]],
            )

    for r in range(RING_ROWS):
        start_row(r, (2 * r, 2 * r + 1))

    def outer_body(o, _):
        base = o * RING_ROWS
        for bb in range(RING_ROWS):
            b = base + bb
            bufs = (2 * bb, 2 * bb + 1)
            acc = tuple(jnp.zeros((LANES,), jnp.float32) for _ in range(VECS))
            for h in range(CHUNKS_PER_ROW):
                buf = bufs[h]
                pltpu.make_async_copy(
                    table_hbm.at[idx_v.at[0]], rows_v.at[buf], sems.at[buf]
                ).wait()

                def acc_body(r, carry, buf=buf):
                    return tuple(
                        carry[k] + rows_v[buf, r, pl.ds(k * LANES, LANES)]
                        for k in range(VECS)
                    )

                acc = lax.fori_loop(0, CHUNK, acc_body, acc, unroll=2)
            for k in range(VECS):
                out_v[b, pl.ds(k * LANES, LANES)] = acc[k] * inv_seq

            @pl.when(b + RING_ROWS < ROWS_PER_W)
            def _():
                start_row(b + RING_ROWS, bufs)

        return 0

    lax.fori_loop(0, ROWS_PER_W // RING_ROWS, outer_body, 0)

    pltpu.sync_copy(out_v, out_hbm.at[pl.ds(wid * ROWS_PER_W, ROWS_PER_W)])


_encoder = pl.kernel(
    _encoder_body,
    out_type=jax.ShapeDtypeStruct((BATCH, EMBED_DIM), jnp.float32),
    mesh=plsc.VectorSubcoreMesh(core_axis_name="c", subcore_axis_name="s"),
    scratch_types=[
        pltpu.VMEM((CHUNKS_PER_W, CHUNK), jnp.int32),
        pltpu.VMEM((NBUF, CHUNK, EMBED_DIM), jnp.float32),
        pltpu.VMEM((ROWS_PER_W, EMBED_DIM), jnp.float32),
        pltpu.SemaphoreType.DMA((NBUF,)),
    ],
    compiler_params=pltpu.CompilerParams(use_tc_tiling_on_sc=False),
)


@jax.jit
def kernel(text_ids, table):
    ids = text_ids.astype(jnp.int32).reshape(NW, CHUNKS_PER_W, CHUNK)
    return _encoder(ids, table)
